# baseline (speedup 1.0000x reference)
import functools

import jax
import jax.numpy as jnp
from jax import lax
from jax.experimental import pallas as pl
from jax.experimental.pallas import tpu as pltpu
from jax.experimental.pallas import tpu_sc as plsc

E = 16
K = 2
M = 256
NEG_INF = -1e30


def _router_body(x_ref, wr_ref, e0_ref, e1_ref, w0_ref, w1_ref, xb16_ref):
    xb = x_ref[...]
    logits = lax.dot_general(xb, wr_ref[...],
                             (((1,), (1,)), ((), ())),
                             preferred_element_type=jnp.float32)
    tb = logits.shape[0]
    iota = lax.broadcasted_iota(jnp.int32, (tb, E), 1)
    m1 = jnp.max(logits, axis=1, keepdims=True)
    i1 = jnp.min(jnp.where(logits == m1, iota, E), axis=1, keepdims=True)
    masked = jnp.where(iota == i1, NEG_INF, logits)
    m2 = jnp.max(masked, axis=1, keepdims=True)
    i2 = jnp.min(jnp.where(masked == m2, iota, E), axis=1, keepdims=True)
    p0 = 1.0 / (1.0 + jnp.exp(m2 - m1))
    e0_ref[...] = i1
    e1_ref[...] = i2
    w0_ref[...] = p0
    w1_ref[...] = 1.0 - p0
    c = xb.shape[1]
    x16 = xb.astype(jnp.bfloat16).reshape(tb, 2, c // 2)
    xb16_ref[...] = pltpu.bitcast(x16, jnp.int32).reshape(tb, c // 2)


def _route(x_flat, Wr):
    n, c = x_flat.shape
    tb = 1024
    grid = (n // tb,)
    outs = pl.pallas_call(
        _router_body,
        grid=grid,
        in_specs=[
            pl.BlockSpec((tb, c), lambda i: (i, 0)),
            pl.BlockSpec((E, c), lambda i: (0, 0)),
        ],
        out_specs=[
            pl.BlockSpec((tb, 1), lambda i: (i, 0)),
            pl.BlockSpec((tb, 1), lambda i: (i, 0)),
            pl.BlockSpec((tb, 1), lambda i: (i, 0)),
            pl.BlockSpec((tb, 1), lambda i: (i, 0)),
            pl.BlockSpec((tb, c // 2), lambda i: (i, 0)),
        ],
        out_shape=[
            jax.ShapeDtypeStruct((n, 1), jnp.int32),
            jax.ShapeDtypeStruct((n, 1), jnp.int32),
            jax.ShapeDtypeStruct((n, 1), jnp.float32),
            jax.ShapeDtypeStruct((n, 1), jnp.float32),
            jax.ShapeDtypeStruct((n, c // 2), jnp.int32),
        ],
    )(x_flat, Wr)
    e0, e1, w0, w1 = (o[:, 0] for o in outs[:4])
    return e0, e1, w0, w1, outs[4]


def _route_metadata(e0, e1, w0, w1, nblk_max):
    n = e0.shape[0]
    na = K * n
    eids = jnp.concatenate([e0, e1]).astype(jnp.int32)
    w_all = jnp.concatenate([w0, w1])
    oh = (eids[:, None] == jnp.arange(E, dtype=jnp.int32)[None, :])
    oh = oh.astype(jnp.int32)
    csum = jnp.cumsum(oh, axis=0)
    counts = csum[-1]
    rank = jnp.take_along_axis(csum - oh, eids[:, None], axis=1)[:, 0]
    nblk = (counts + M - 1) // M
    cum_nblk = jnp.cumsum(nblk)
    total = cum_nblk[-1]
    blk_start = cum_nblk - nblk
    pad_off = blk_start * M
    padded_pos = pad_off[eids] + rank
    npad = nblk_max * M
    scatter_ids = jnp.zeros((npad,), jnp.int32).at[padded_pos].set(
        jnp.arange(na, dtype=jnp.int32))
    gather_idx = scatter_ids % n
    w_sorted = w_all[scatter_ids]
    bb = jnp.arange(nblk_max, dtype=jnp.int32)
    blk_exp = jnp.searchsorted(cum_nblk, bb, side="right").astype(jnp.int32)
    last_exp = blk_exp[total - 1]
    gmap = jnp.where(bb < total, blk_exp, last_exp)
    xmap = jnp.where(bb < total, bb, total - 1)
    active = (bb < total).astype(jnp.int32)
    p0 = padded_pos[:n]
    p1 = padded_pos[n:]
    return gather_idx, w_sorted, gmap, xmap, active, p0, p1


def _gmm_body(gmap_ref, xmap_ref, active_ref, wvec_ref, x_ref, w1_ref, w2_ref,
              out_ref):
    b = pl.program_id(0)

    @pl.when(active_ref[b] == 1)
    def _():
        xi = x_ref[...]
        m, c2 = xi.shape
        xb = pltpu.bitcast(xi.reshape(m, 1, c2),
                           jnp.bfloat16).reshape(m, 2 * c2)
        h = lax.dot_general(xb, w1_ref[0],
                            (((1,), (1,)), ((), ())),
                            preferred_element_type=jnp.float32)
        h = jnp.square(jnp.maximum(h, 0.0)).astype(jnp.bfloat16)
        y = lax.dot_general(h, w2_ref[0],
                            (((1,), (1,)), ((), ())),
                            preferred_element_type=jnp.float32)
        y = (y * wvec_ref[...]).astype(jnp.bfloat16)
        out_ref[...] = pltpu.bitcast(
            y.reshape(m, 2, c2), jnp.int32).reshape(m, c2)


def _gmm(x_sorted, w_sorted, W1, W2, gmap, xmap, active, nblk_max):
    npad, c2 = x_sorted.shape
    d, c = W1.shape[1], W1.shape[2]
    grid_spec = pltpu.PrefetchScalarGridSpec(
        num_scalar_prefetch=3,
        grid=(nblk_max,),
        in_specs=[
            pl.BlockSpec((M, 1), lambda b, g, xm, a: (xm[b], 0)),
            pl.BlockSpec((M, c2), lambda b, g, xm, a: (xm[b], 0)),
            pl.BlockSpec((1, d, c), lambda b, g, xm, a: (g[b], 0, 0)),
            pl.BlockSpec((1, c, d), lambda b, g, xm, a: (g[b], 0, 0)),
        ],
        out_specs=pl.BlockSpec((M, c2), lambda b, g, xm, a: (xm[b], 0)),
    )
    return pl.pallas_call(
        _gmm_body,
        grid_spec=grid_spec,
        out_shape=jax.ShapeDtypeStruct((npad, c2), jnp.int32),
    )(gmap, xmap, active, w_sorted[:, None], x_sorted, W1, W2)


_NW = 32
_CH = 16
_NBUF = 5


def _gather_rows(x_flat, gather_idx):
    n, c = x_flat.shape
    dt = x_flat.dtype
    npad = gather_idx.shape[0]
    rows_per_w = npad // _NW
    nch = rows_per_w // _CH
    mesh = plsc.VectorSubcoreMesh(core_axis_name="c", subcore_axis_name="s")

    @functools.partial(
        pl.kernel,
        out_type=jax.ShapeDtypeStruct((npad, c), dt),
        mesh=mesh,
        scratch_types=[
            pltpu.VMEM((rows_per_w,), jnp.int32),
            [pltpu.VMEM((_CH, c), dt) for _ in range(_NBUF)],
            [pltpu.SemaphoreType.DMA for _ in range(_NBUF)],
            [pltpu.SemaphoreType.DMA for _ in range(_NBUF)],
        ],
    )
    def gather_k(x_hbm, idx_hbm, out_hbm, idx_v, bufs, gsems, ssems):
        wid = lax.axis_index("s") * 2 + lax.axis_index("c")
        base = wid * rows_per_w
        pltpu.sync_copy(idx_hbm.at[pl.ds(base, rows_per_w)], idx_v)

        def g_copy(ch, b):
            return pltpu.make_async_copy(
                x_hbm.at[idx_v.at[pl.ds(ch * _CH, _CH)]], bufs[b], gsems[b])

        def s_copy(ch, b):
            return pltpu.make_async_copy(
                bufs[b], out_hbm.at[pl.ds(base + ch * _CH, _CH)], ssems[b])

        def body(i, carry):
            for b in range(_NBUF):
                g_copy(i * _NBUF + b, b).start()
            for b in range(_NBUF):
                g_copy(i * _NBUF + b, b).wait()
                s_copy(i * _NBUF + b, b).start()
            for b in range(_NBUF):
                s_copy(i * _NBUF + b, b).wait()
            return carry

        lax.fori_loop(0, nch // _NBUF, body, 0)

    return gather_k(x_flat, gather_idx)


def _combine_gather(out_sorted, p0, p1):
    npad, c2 = out_sorted.shape
    n = p0.shape[0]
    rows_per_w = n // _NW
    nch = rows_per_w // _CH
    mesh = plsc.VectorSubcoreMesh(core_axis_name="c", subcore_axis_name="s")

    @functools.partial(
        pl.kernel,
        out_type=(jax.ShapeDtypeStruct((n, c2), jnp.int32),
                  jax.ShapeDtypeStruct((n, c2), jnp.int32)),
        mesh=mesh,
        scratch_types=[
            pltpu.VMEM((rows_per_w,), jnp.int32),
            pltpu.VMEM((rows_per_w,), jnp.int32),
            [pltpu.VMEM((_CH, c2), jnp.int32) for _ in range(4)],
            [pltpu.SemaphoreType.DMA for _ in range(4)],
            [pltpu.SemaphoreType.DMA for _ in range(4)],
        ],
    )
    def combine_k(src_hbm, p0_hbm, p1_hbm, a_hbm, b_hbm, i0_v, i1_v, bufs,
                  gsems, ssems):
        wid = lax.axis_index("s") * 2 + lax.axis_index("c")
        base = wid * rows_per_w
        pltpu.sync_copy(p0_hbm.at[pl.ds(base, rows_per_w)], i0_v)
        pltpu.sync_copy(p1_hbm.at[pl.ds(base, rows_per_w)], i1_v)

        def body(i, carry):
            for u in range(2):
                ch = i * 2 + u
                for k in range(2):
                    idx_v = (i0_v, i1_v)[k]
                    b = 2 * u + k
                    pltpu.make_async_copy(
                        src_hbm.at[idx_v.at[pl.ds(ch * _CH, _CH)]],
                        bufs[b], gsems[b]).start()
            for u in range(2):
                ch = i * 2 + u
                for k in range(2):
                    idx_v = (i0_v, i1_v)[k]
                    dst = (a_hbm, b_hbm)[k]
                    b = 2 * u + k
                    pltpu.make_async_copy(
                        src_hbm.at[idx_v.at[pl.ds(ch * _CH, _CH)]],
                        bufs[b], gsems[b]).wait()
                    pltpu.make_async_copy(
                        bufs[b], dst.at[pl.ds(base + ch * _CH, _CH)],
                        ssems[b]).start()
            for b in range(4):
                u, k = divmod(b, 2)
                ch = i * 2 + u
                dst = (a_hbm, b_hbm)[k]
                pltpu.make_async_copy(
                    bufs[b], dst.at[pl.ds(base + ch * _CH, _CH)],
                    ssems[b]).wait()
            return carry

        lax.fori_loop(0, nch // 2, body, 0)

    return combine_k(out_sorted, p0, p1)


def _add_body(a_ref, b_ref, out_ref):
    m, c2 = a_ref.shape
    a = pltpu.bitcast(a_ref[...].reshape(m, 1, c2), jnp.bfloat16)
    b = pltpu.bitcast(b_ref[...].reshape(m, 1, c2), jnp.bfloat16)
    s = a.astype(jnp.float32) + b.astype(jnp.float32)
    out_ref[...] = s.reshape(m, 2 * c2)


def _final_add(a, b):
    n, c2 = a.shape
    tb = 512
    return pl.pallas_call(
        _add_body,
        grid=(n // tb,),
        in_specs=[
            pl.BlockSpec((tb, c2), lambda i: (i, 0)),
            pl.BlockSpec((tb, c2), lambda i: (i, 0)),
        ],
        out_specs=pl.BlockSpec((tb, 2 * c2), lambda i: (i, 0)),
        out_shape=jax.ShapeDtypeStruct((n, 2 * c2), jnp.float32),
    )(a, b)


def kernel(x, Wr, W1, W2):
    bx, t, c = x.shape
    n = bx * t
    nblk_max = (K * n) // M + E
    x_flat = x.reshape(n, c)
    e0, e1, w0, w1, x16p = _route(x_flat, Wr)
    gather_idx, w_sorted, gmap, xmap, active, p0, p1 = _route_metadata(
        e0, e1, w0, w1, nblk_max)
    x_sorted = _gather_rows(x16p, gather_idx)
    out_sorted = _gmm(x_sorted, w_sorted, W1.astype(jnp.bfloat16),
                      W2.astype(jnp.bfloat16), gmap, xmap, active, nblk_max)
    a, b = _combine_gather(out_sorted, p0, p1)
    out = _final_add(a, b)
    return out.reshape(bx, t, c)


# speedup vs baseline: 1.0597x; 1.0597x over previous
import functools

import jax
import jax.numpy as jnp
from jax import lax
from jax.experimental import pallas as pl
from jax.experimental.pallas import tpu as pltpu
from jax.experimental.pallas import tpu_sc as plsc

E = 16
K = 2
M = 256
NEG_INF = -1e30


def _router_body(x_ref, wr_ref, e0_ref, e1_ref, w0_ref, w1_ref, xb16_ref):
    xb = x_ref[...]
    logits = lax.dot_general(xb, wr_ref[...],
                             (((1,), (1,)), ((), ())),
                             preferred_element_type=jnp.float32)
    tb = logits.shape[0]
    iota = lax.broadcasted_iota(jnp.int32, (tb, E), 1)
    m1 = jnp.max(logits, axis=1, keepdims=True)
    i1 = jnp.min(jnp.where(logits == m1, iota, E), axis=1, keepdims=True)
    masked = jnp.where(iota == i1, NEG_INF, logits)
    m2 = jnp.max(masked, axis=1, keepdims=True)
    i2 = jnp.min(jnp.where(masked == m2, iota, E), axis=1, keepdims=True)
    p0 = 1.0 / (1.0 + jnp.exp(m2 - m1))
    e0_ref[...] = i1
    e1_ref[...] = i2
    w0_ref[...] = p0
    w1_ref[...] = 1.0 - p0
    c = xb.shape[1]
    x16 = xb.astype(jnp.bfloat16).reshape(tb, 2, c // 2)
    xb16_ref[...] = pltpu.bitcast(x16, jnp.int32).reshape(tb, c // 2)


def _route(x_flat, Wr):
    n, c = x_flat.shape
    tb = 1024
    grid = (n // tb,)
    outs = pl.pallas_call(
        _router_body,
        grid=grid,
        in_specs=[
            pl.BlockSpec((tb, c), lambda i: (i, 0)),
            pl.BlockSpec((E, c), lambda i: (0, 0)),
        ],
        out_specs=[
            pl.BlockSpec((tb, 1), lambda i: (i, 0)),
            pl.BlockSpec((tb, 1), lambda i: (i, 0)),
            pl.BlockSpec((tb, 1), lambda i: (i, 0)),
            pl.BlockSpec((tb, 1), lambda i: (i, 0)),
            pl.BlockSpec((tb, c // 2), lambda i: (i, 0)),
        ],
        out_shape=[
            jax.ShapeDtypeStruct((n, 1), jnp.int32),
            jax.ShapeDtypeStruct((n, 1), jnp.int32),
            jax.ShapeDtypeStruct((n, 1), jnp.float32),
            jax.ShapeDtypeStruct((n, 1), jnp.float32),
            jax.ShapeDtypeStruct((n, c // 2), jnp.int32),
        ],
    )(x_flat, Wr)
    e0, e1, w0, w1 = (o[:, 0] for o in outs[:4])
    return e0, e1, w0, w1, outs[4]


def _route_metadata(e0, e1, w0, w1, nblk_max):
    n = e0.shape[0]
    na = K * n
    eids = jnp.concatenate([e0, e1]).astype(jnp.int32)
    w_all = jnp.concatenate([w0, w1])
    oh = (eids[:, None] == jnp.arange(E, dtype=jnp.int32)[None, :])
    oh = oh.astype(jnp.int32)
    csum = jnp.cumsum(oh, axis=0)
    counts = csum[-1]
    rank = jnp.take_along_axis(csum - oh, eids[:, None], axis=1)[:, 0]
    nblk = (counts + M - 1) // M
    cum_nblk = jnp.cumsum(nblk)
    total = cum_nblk[-1]
    blk_start = cum_nblk - nblk
    pad_off = blk_start * M
    padded_pos = pad_off[eids] + rank
    npad = nblk_max * M
    scatter_ids = jnp.zeros((npad,), jnp.int32).at[padded_pos].set(
        jnp.arange(na, dtype=jnp.int32))
    gather_idx = scatter_ids % n
    w_sorted = w_all[scatter_ids]
    bb = jnp.arange(nblk_max, dtype=jnp.int32)
    blk_exp = jnp.searchsorted(cum_nblk, bb, side="right").astype(jnp.int32)
    last_exp = blk_exp[total - 1]
    gmap = jnp.where(bb < total, blk_exp, last_exp)
    xmap = jnp.where(bb < total, bb, total - 1)
    active = (bb < total).astype(jnp.int32)
    p0 = padded_pos[:n]
    p1 = padded_pos[n:]
    return gather_idx, w_sorted, gmap, xmap, active, p0, p1


def _gmm_body(gmap_ref, xmap_ref, active_ref, wvec_ref, x_ref, w1_ref, w2_ref,
              out_ref, w1s, w2s):
    b = pl.program_id(0)
    prev_g = jnp.where(b > 0, gmap_ref[jnp.maximum(b - 1, 0)], -1)

    @pl.when(prev_g != gmap_ref[b])
    def _():
        w1s[...] = w1_ref[0].astype(jnp.bfloat16)
        w2s[...] = w2_ref[0].astype(jnp.bfloat16)

    @pl.when(active_ref[b] == 1)
    def _():
        xi = x_ref[...]
        m, c2 = xi.shape
        xb = pltpu.bitcast(xi.reshape(m, 1, c2),
                           jnp.bfloat16).reshape(m, 2 * c2)
        h = lax.dot_general(xb, w1s[...],
                            (((1,), (1,)), ((), ())),
                            preferred_element_type=jnp.float32)
        h = jnp.square(jnp.maximum(h, 0.0)).astype(jnp.bfloat16)
        y = lax.dot_general(h, w2s[...],
                            (((1,), (1,)), ((), ())),
                            preferred_element_type=jnp.float32)
        y = (y * wvec_ref[...]).astype(jnp.bfloat16)
        out_ref[...] = pltpu.bitcast(
            y.reshape(m, 2, c2), jnp.int32).reshape(m, c2)


def _gmm(x_sorted, w_sorted, W1, W2, gmap, xmap, active, nblk_max):
    npad, c2 = x_sorted.shape
    d, c = W1.shape[1], W1.shape[2]
    grid_spec = pltpu.PrefetchScalarGridSpec(
        num_scalar_prefetch=3,
        grid=(nblk_max,),
        in_specs=[
            pl.BlockSpec((M, 1), lambda b, g, xm, a: (xm[b], 0)),
            pl.BlockSpec((M, c2), lambda b, g, xm, a: (xm[b], 0)),
            pl.BlockSpec((1, d, c), lambda b, g, xm, a: (g[b], 0, 0)),
            pl.BlockSpec((1, c, d), lambda b, g, xm, a: (g[b], 0, 0)),
        ],
        out_specs=pl.BlockSpec((M, c2), lambda b, g, xm, a: (xm[b], 0)),
        scratch_shapes=[
            pltpu.VMEM((d, c), jnp.bfloat16),
            pltpu.VMEM((c, d), jnp.bfloat16),
        ],
    )
    return pl.pallas_call(
        _gmm_body,
        grid_spec=grid_spec,
        out_shape=jax.ShapeDtypeStruct((npad, c2), jnp.int32),
    )(gmap, xmap, active, w_sorted[:, None], x_sorted, W1, W2)


_NW = 32
_CH = 16
_NBUF = 5


def _gather_rows(x_flat, gather_idx):
    n, c = x_flat.shape
    dt = x_flat.dtype
    npad = gather_idx.shape[0]
    rows_per_w = npad // _NW
    nch = rows_per_w // _CH
    mesh = plsc.VectorSubcoreMesh(core_axis_name="c", subcore_axis_name="s")

    @functools.partial(
        pl.kernel,
        out_type=jax.ShapeDtypeStruct((npad, c), dt),
        mesh=mesh,
        scratch_types=[
            pltpu.VMEM((rows_per_w,), jnp.int32),
            [pltpu.VMEM((_CH, c), dt) for _ in range(_NBUF)],
            [pltpu.SemaphoreType.DMA for _ in range(_NBUF)],
            [pltpu.SemaphoreType.DMA for _ in range(_NBUF)],
        ],
    )
    def gather_k(x_hbm, idx_hbm, out_hbm, idx_v, bufs, gsems, ssems):
        wid = lax.axis_index("s") * 2 + lax.axis_index("c")
        base = wid * rows_per_w
        pltpu.sync_copy(idx_hbm.at[pl.ds(base, rows_per_w)], idx_v)

        def g_copy(ch, b):
            return pltpu.make_async_copy(
                x_hbm.at[idx_v.at[pl.ds(ch * _CH, _CH)]], bufs[b], gsems[b])

        def s_copy(ch, b):
            return pltpu.make_async_copy(
                bufs[b], out_hbm.at[pl.ds(base + ch * _CH, _CH)], ssems[b])

        def body(i, carry):
            for b in range(_NBUF):
                g_copy(i * _NBUF + b, b).start()
            for b in range(_NBUF):
                g_copy(i * _NBUF + b, b).wait()
                s_copy(i * _NBUF + b, b).start()
            for b in range(_NBUF):
                s_copy(i * _NBUF + b, b).wait()
            return carry

        lax.fori_loop(0, nch // _NBUF, body, 0)

    return gather_k(x_flat, gather_idx)


def _combine_gather(out_sorted, p0, p1):
    npad, c2 = out_sorted.shape
    n = p0.shape[0]
    rows_per_w = n // _NW
    nch = rows_per_w // _CH
    mesh = plsc.VectorSubcoreMesh(core_axis_name="c", subcore_axis_name="s")

    @functools.partial(
        pl.kernel,
        out_type=(jax.ShapeDtypeStruct((n, c2), jnp.int32),
                  jax.ShapeDtypeStruct((n, c2), jnp.int32)),
        mesh=mesh,
        scratch_types=[
            pltpu.VMEM((rows_per_w,), jnp.int32),
            pltpu.VMEM((rows_per_w,), jnp.int32),
            [pltpu.VMEM((_CH, c2), jnp.int32) for _ in range(4)],
            [pltpu.SemaphoreType.DMA for _ in range(4)],
            [pltpu.SemaphoreType.DMA for _ in range(4)],
        ],
    )
    def combine_k(src_hbm, p0_hbm, p1_hbm, a_hbm, b_hbm, i0_v, i1_v, bufs,
                  gsems, ssems):
        wid = lax.axis_index("s") * 2 + lax.axis_index("c")
        base = wid * rows_per_w
        pltpu.sync_copy(p0_hbm.at[pl.ds(base, rows_per_w)], i0_v)
        pltpu.sync_copy(p1_hbm.at[pl.ds(base, rows_per_w)], i1_v)

        def body(i, carry):
            for u in range(2):
                ch = i * 2 + u
                for k in range(2):
                    idx_v = (i0_v, i1_v)[k]
                    b = 2 * u + k
                    pltpu.make_async_copy(
                        src_hbm.at[idx_v.at[pl.ds(ch * _CH, _CH)]],
                        bufs[b], gsems[b]).start()
            for u in range(2):
                ch = i * 2 + u
                for k in range(2):
                    idx_v = (i0_v, i1_v)[k]
                    dst = (a_hbm, b_hbm)[k]
                    b = 2 * u + k
                    pltpu.make_async_copy(
                        src_hbm.at[idx_v.at[pl.ds(ch * _CH, _CH)]],
                        bufs[b], gsems[b]).wait()
                    pltpu.make_async_copy(
                        bufs[b], dst.at[pl.ds(base + ch * _CH, _CH)],
                        ssems[b]).start()
            for b in range(4):
                u, k = divmod(b, 2)
                ch = i * 2 + u
                dst = (a_hbm, b_hbm)[k]
                pltpu.make_async_copy(
                    bufs[b], dst.at[pl.ds(base + ch * _CH, _CH)],
                    ssems[b]).wait()
            return carry

        lax.fori_loop(0, nch // 2, body, 0)

    return combine_k(out_sorted, p0, p1)


def _add_body(a_ref, b_ref, out_ref):
    m, c2 = a_ref.shape
    a = pltpu.bitcast(a_ref[...].reshape(m, 1, c2), jnp.bfloat16)
    b = pltpu.bitcast(b_ref[...].reshape(m, 1, c2), jnp.bfloat16)
    s = a.astype(jnp.float32) + b.astype(jnp.float32)
    out_ref[...] = s.reshape(m, 2 * c2)


def _final_add(a, b):
    n, c2 = a.shape
    tb = 512
    return pl.pallas_call(
        _add_body,
        grid=(n // tb,),
        in_specs=[
            pl.BlockSpec((tb, c2), lambda i: (i, 0)),
            pl.BlockSpec((tb, c2), lambda i: (i, 0)),
        ],
        out_specs=pl.BlockSpec((tb, 2 * c2), lambda i: (i, 0)),
        out_shape=jax.ShapeDtypeStruct((n, 2 * c2), jnp.float32),
    )(a, b)


def kernel(x, Wr, W1, W2):
    bx, t, c = x.shape
    n = bx * t
    nblk_max = (K * n) // M + E
    x_flat = x.reshape(n, c)
    e0, e1, w0, w1, x16p = _route(x_flat, Wr)
    gather_idx, w_sorted, gmap, xmap, active, p0, p1 = _route_metadata(
        e0, e1, w0, w1, nblk_max)
    x_sorted = _gather_rows(x16p, gather_idx)
    out_sorted = _gmm(x_sorted, w_sorted, W1, W2, gmap, xmap, active, nblk_max)
    a, b = _combine_gather(out_sorted, p0, p1)
    out = _final_add(a, b)
    return out.reshape(bx, t, c)


# speedup vs baseline: 1.2572x; 1.1863x over previous
import functools

import jax
import jax.numpy as jnp
from jax import lax
from jax.experimental import pallas as pl
from jax.experimental.pallas import tpu as pltpu
from jax.experimental.pallas import tpu_sc as plsc

E = 16
K = 2
M = 256
NEG_INF = -1e30


def _router_body(x_ref, wr_ref, e0_ref, e1_ref, w0_ref, w1_ref, xb16_ref):
    xb = x_ref[...]
    logits = lax.dot_general(xb, wr_ref[...],
                             (((1,), (1,)), ((), ())),
                             preferred_element_type=jnp.float32)
    tb = logits.shape[0]
    iota = lax.broadcasted_iota(jnp.int32, (tb, E), 1)
    m1 = jnp.max(logits, axis=1, keepdims=True)
    i1 = jnp.min(jnp.where(logits == m1, iota, E), axis=1, keepdims=True)
    masked = jnp.where(iota == i1, NEG_INF, logits)
    m2 = jnp.max(masked, axis=1, keepdims=True)
    i2 = jnp.min(jnp.where(masked == m2, iota, E), axis=1, keepdims=True)
    p0 = 1.0 / (1.0 + jnp.exp(m2 - m1))
    e0_ref[...] = i1
    e1_ref[...] = i2
    w0_ref[...] = p0
    w1_ref[...] = 1.0 - p0
    c2 = xb.shape[1] // 2
    xb16_ref[...] = pltpu.pack_elementwise(
        [xb[:, :c2], xb[:, c2:]], packed_dtype=jnp.bfloat16)


def _route(x_flat, Wr):
    n, c = x_flat.shape
    tb = 1024
    grid = (n // tb,)
    outs = pl.pallas_call(
        _router_body,
        grid=grid,
        in_specs=[
            pl.BlockSpec((tb, c), lambda i: (i, 0)),
            pl.BlockSpec((E, c), lambda i: (0, 0)),
        ],
        out_specs=[
            pl.BlockSpec((tb, 1), lambda i: (i, 0)),
            pl.BlockSpec((tb, 1), lambda i: (i, 0)),
            pl.BlockSpec((tb, 1), lambda i: (i, 0)),
            pl.BlockSpec((tb, 1), lambda i: (i, 0)),
            pl.BlockSpec((tb, c // 2), lambda i: (i, 0)),
        ],
        out_shape=[
            jax.ShapeDtypeStruct((n, 1), jnp.int32),
            jax.ShapeDtypeStruct((n, 1), jnp.int32),
            jax.ShapeDtypeStruct((n, 1), jnp.float32),
            jax.ShapeDtypeStruct((n, 1), jnp.float32),
            jax.ShapeDtypeStruct((n, c // 2), jnp.int32),
        ],
    )(x_flat, Wr)
    e0, e1, w0, w1 = (o[:, 0] for o in outs[:4])
    return e0, e1, w0, w1, outs[4]


def _route_metadata(e0, e1, w0, w1, nblk_max):
    n = e0.shape[0]
    na = K * n
    eids = jnp.concatenate([e0, e1]).astype(jnp.int32)
    w_all = jnp.concatenate([w0, w1])
    oh = (eids[:, None] == jnp.arange(E, dtype=jnp.int32)[None, :])
    oh = oh.astype(jnp.int32)
    csum = jnp.cumsum(oh, axis=0)
    counts = csum[-1]
    rank = jnp.take_along_axis(csum - oh, eids[:, None], axis=1)[:, 0]
    nblk = (counts + M - 1) // M
    cum_nblk = jnp.cumsum(nblk)
    total = cum_nblk[-1]
    blk_start = cum_nblk - nblk
    pad_off = blk_start * M
    padded_pos = pad_off[eids] + rank
    npad = nblk_max * M
    scatter_ids = jnp.zeros((npad,), jnp.int32).at[padded_pos].set(
        jnp.arange(na, dtype=jnp.int32))
    gather_idx = scatter_ids % n
    w_sorted = w_all[scatter_ids]
    bb = jnp.arange(nblk_max, dtype=jnp.int32)
    blk_exp = jnp.searchsorted(cum_nblk, bb, side="right").astype(jnp.int32)
    last_exp = blk_exp[total - 1]
    gmap = jnp.where(bb < total, blk_exp, last_exp)
    xmap = jnp.where(bb < total, bb, total - 1)
    active = (bb < total).astype(jnp.int32)
    p0 = padded_pos[:n]
    p1 = padded_pos[n:]
    return gather_idx, w_sorted, gmap, xmap, active, p0, p1


def _gmm_body(gmap_ref, xmap_ref, active_ref, wvec_ref, x_ref, w1_ref, w2_ref,
              out_ref, w1s, w2s):
    b = pl.program_id(0)
    prev_g = jnp.where(b > 0, gmap_ref[jnp.maximum(b - 1, 0)], -1)

    @pl.when(prev_g != gmap_ref[b])
    def _():
        w1s[...] = w1_ref[0].astype(jnp.bfloat16)
        w2s[...] = w2_ref[0].astype(jnp.bfloat16)

    @pl.when(active_ref[b] == 1)
    def _():
        xi = x_ref[...]
        m, c2 = xi.shape
        xlo = pltpu.unpack_elementwise(
            xi, index=0, packed_dtype=jnp.bfloat16,
            unpacked_dtype=jnp.float32).astype(jnp.bfloat16)
        xhi = pltpu.unpack_elementwise(
            xi, index=1, packed_dtype=jnp.bfloat16,
            unpacked_dtype=jnp.float32).astype(jnp.bfloat16)
        cd = (((1,), (1,)), ((), ()))
        h = (lax.dot_general(xlo, w1s[:, :c2], cd,
                             preferred_element_type=jnp.float32)
             + lax.dot_general(xhi, w1s[:, c2:], cd,
                               preferred_element_type=jnp.float32))
        h = jnp.square(jnp.maximum(h, 0.0)).astype(jnp.bfloat16)
        y = lax.dot_general(h, w2s[...], cd,
                            preferred_element_type=jnp.float32)
        y = y * wvec_ref[...]
        out_ref[...] = pltpu.pack_elementwise(
            [y[:, :c2], y[:, c2:]], packed_dtype=jnp.bfloat16)


def _gmm(x_sorted, w_sorted, W1, W2, gmap, xmap, active, nblk_max):
    npad, c2 = x_sorted.shape
    d, c = W1.shape[1], W1.shape[2]
    grid_spec = pltpu.PrefetchScalarGridSpec(
        num_scalar_prefetch=3,
        grid=(nblk_max,),
        in_specs=[
            pl.BlockSpec((M, 1), lambda b, g, xm, a: (xm[b], 0)),
            pl.BlockSpec((M, c2), lambda b, g, xm, a: (xm[b], 0)),
            pl.BlockSpec((1, d, c), lambda b, g, xm, a: (g[b], 0, 0)),
            pl.BlockSpec((1, c, d), lambda b, g, xm, a: (g[b], 0, 0)),
        ],
        out_specs=pl.BlockSpec((M, c2), lambda b, g, xm, a: (xm[b], 0)),
        scratch_shapes=[
            pltpu.VMEM((d, c), jnp.bfloat16),
            pltpu.VMEM((c, d), jnp.bfloat16),
        ],
    )
    return pl.pallas_call(
        _gmm_body,
        grid_spec=grid_spec,
        out_shape=jax.ShapeDtypeStruct((npad, c2), jnp.int32),
    )(gmap, xmap, active, w_sorted[:, None], x_sorted, W1, W2)


_NW = 32
_CH = 16
_NBUF = 5


def _gather_rows(x_flat, gather_idx):
    n, c = x_flat.shape
    dt = x_flat.dtype
    npad = gather_idx.shape[0]
    rows_per_w = npad // _NW
    nch = rows_per_w // _CH
    mesh = plsc.VectorSubcoreMesh(core_axis_name="c", subcore_axis_name="s")

    @functools.partial(
        pl.kernel,
        out_type=jax.ShapeDtypeStruct((npad, c), dt),
        mesh=mesh,
        scratch_types=[
            pltpu.VMEM((rows_per_w,), jnp.int32),
            [pltpu.VMEM((_CH, c), dt) for _ in range(_NBUF)],
            [pltpu.SemaphoreType.DMA for _ in range(_NBUF)],
            [pltpu.SemaphoreType.DMA for _ in range(_NBUF)],
        ],
    )
    def gather_k(x_hbm, idx_hbm, out_hbm, idx_v, bufs, gsems, ssems):
        wid = lax.axis_index("s") * 2 + lax.axis_index("c")
        base = wid * rows_per_w
        pltpu.sync_copy(idx_hbm.at[pl.ds(base, rows_per_w)], idx_v)

        def g_copy(ch, b):
            return pltpu.make_async_copy(
                x_hbm.at[idx_v.at[pl.ds(ch * _CH, _CH)]], bufs[b], gsems[b])

        def s_copy(ch, b):
            return pltpu.make_async_copy(
                bufs[b], out_hbm.at[pl.ds(base + ch * _CH, _CH)], ssems[b])

        def body(i, carry):
            for b in range(_NBUF):
                g_copy(i * _NBUF + b, b).start()
            for b in range(_NBUF):
                g_copy(i * _NBUF + b, b).wait()
                s_copy(i * _NBUF + b, b).start()
            for b in range(_NBUF):
                s_copy(i * _NBUF + b, b).wait()
            return carry

        lax.fori_loop(0, nch // _NBUF, body, 0)

    return gather_k(x_flat, gather_idx)


def _combine_gather(out_sorted, p0, p1):
    npad, c2 = out_sorted.shape
    n = p0.shape[0]
    rows_per_w = n // _NW
    nch = rows_per_w // _CH
    mesh = plsc.VectorSubcoreMesh(core_axis_name="c", subcore_axis_name="s")

    @functools.partial(
        pl.kernel,
        out_type=(jax.ShapeDtypeStruct((n, c2), jnp.int32),
                  jax.ShapeDtypeStruct((n, c2), jnp.int32)),
        mesh=mesh,
        scratch_types=[
            pltpu.VMEM((rows_per_w,), jnp.int32),
            pltpu.VMEM((rows_per_w,), jnp.int32),
            [pltpu.VMEM((_CH, c2), jnp.int32) for _ in range(4)],
            [pltpu.SemaphoreType.DMA for _ in range(4)],
            [pltpu.SemaphoreType.DMA for _ in range(4)],
        ],
    )
    def combine_k(src_hbm, p0_hbm, p1_hbm, a_hbm, b_hbm, i0_v, i1_v, bufs,
                  gsems, ssems):
        wid = lax.axis_index("s") * 2 + lax.axis_index("c")
        base = wid * rows_per_w
        pltpu.sync_copy(p0_hbm.at[pl.ds(base, rows_per_w)], i0_v)
        pltpu.sync_copy(p1_hbm.at[pl.ds(base, rows_per_w)], i1_v)

        def body(i, carry):
            for u in range(2):
                ch = i * 2 + u
                for k in range(2):
                    idx_v = (i0_v, i1_v)[k]
                    b = 2 * u + k
                    pltpu.make_async_copy(
                        src_hbm.at[idx_v.at[pl.ds(ch * _CH, _CH)]],
                        bufs[b], gsems[b]).start()
            for u in range(2):
                ch = i * 2 + u
                for k in range(2):
                    idx_v = (i0_v, i1_v)[k]
                    dst = (a_hbm, b_hbm)[k]
                    b = 2 * u + k
                    pltpu.make_async_copy(
                        src_hbm.at[idx_v.at[pl.ds(ch * _CH, _CH)]],
                        bufs[b], gsems[b]).wait()
                    pltpu.make_async_copy(
                        bufs[b], dst.at[pl.ds(base + ch * _CH, _CH)],
                        ssems[b]).start()
            for b in range(4):
                u, k = divmod(b, 2)
                ch = i * 2 + u
                dst = (a_hbm, b_hbm)[k]
                pltpu.make_async_copy(
                    bufs[b], dst.at[pl.ds(base + ch * _CH, _CH)],
                    ssems[b]).wait()
            return carry

        lax.fori_loop(0, nch // 2, body, 0)

    return combine_k(out_sorted, p0, p1)


def _add_body(a_ref, b_ref, out_ref):
    m, c2 = a_ref.shape
    ai, bi = a_ref[...], b_ref[...]

    def half(k):
        a = pltpu.unpack_elementwise(ai, index=k, packed_dtype=jnp.bfloat16,
                                     unpacked_dtype=jnp.float32)
        b = pltpu.unpack_elementwise(bi, index=k, packed_dtype=jnp.bfloat16,
                                     unpacked_dtype=jnp.float32)
        return a + b

    out_ref[:, :c2] = half(0)
    out_ref[:, c2:] = half(1)


def _final_add(a, b):
    n, c2 = a.shape
    tb = 512
    return pl.pallas_call(
        _add_body,
        grid=(n // tb,),
        in_specs=[
            pl.BlockSpec((tb, c2), lambda i: (i, 0)),
            pl.BlockSpec((tb, c2), lambda i: (i, 0)),
        ],
        out_specs=pl.BlockSpec((tb, 2 * c2), lambda i: (i, 0)),
        out_shape=jax.ShapeDtypeStruct((n, 2 * c2), jnp.float32),
    )(a, b)


def kernel(x, Wr, W1, W2):
    bx, t, c = x.shape
    n = bx * t
    nblk_max = (K * n) // M + E
    x_flat = x.reshape(n, c)
    e0, e1, w0, w1, x16p = _route(x_flat, Wr)
    gather_idx, w_sorted, gmap, xmap, active, p0, p1 = _route_metadata(
        e0, e1, w0, w1, nblk_max)
    x_sorted = _gather_rows(x16p, gather_idx)
    out_sorted = _gmm(x_sorted, w_sorted, W1, W2, gmap, xmap, active, nblk_max)
    a, b = _combine_gather(out_sorted, p0, p1)
    out = _final_add(a, b)
    return out.reshape(bx, t, c)
